# Initial kernel scaffold; baseline (speedup 1.0000x reference)
import jax
import jax.numpy as jnp
from jax.experimental import pallas as pl


def kernel(s0, s1, s2, s3, y):
    raise NotImplementedError("write your pallas kernel here")


# speedup vs baseline: 1.4188x; 1.4188x over previous
import functools

import jax
import jax.numpy as jnp
from jax.experimental import pallas as pl
from jax.experimental.pallas import tpu as pltpu

BN = 2048
SLOTS = 16
GROUP = 128 // SLOTS
K = 10
INF = float("inf")
BIGI = 2**30


def _dist_topk_kernel(a_ref, y_ref, out_ref, cval_ref, cidx_ref,
                      bufv_ref, bufi_ref, *, n, nb):
    j = pl.program_id(0)
    q = a_ref.shape[0]

    @pl.when(j % GROUP == 0)
    def _init_buf():
        bufv_ref[...] = jnp.full(bufv_ref.shape, INF, jnp.float32)
        bufi_ref[...] = jnp.full(bufi_ref.shape, BIGI, jnp.int32)

    a = a_ref[...]
    yb = y_ref[...]
    ab = jax.lax.dot_general(
        a.astype(jnp.bfloat16), yb.astype(jnp.bfloat16),
        (((1,), (1,)), ((), ())),
        preferred_element_type=jnp.float32)
    a2 = jnp.sum(a * a, axis=1, keepdims=True)
    b2 = jnp.sum(yb * yb, axis=1)[None, :]
    d = (a2 + b2) - 2.0 * ab
    iota = jax.lax.broadcasted_iota(jnp.int32, (q, BN), 1)
    d = jnp.where(iota < (n - j * BN), d, INF)

    lane = jax.lax.broadcasted_iota(jnp.int32, (q, 128), 1)
    slot0 = (j % GROUP) * SLOTS

    def body(t, carry):
        dcur, bv, bi = carry
        m = jnp.min(dcur, axis=1, keepdims=True)
        idx = jnp.min(jnp.where(dcur == m, iota, BIGI), axis=1, keepdims=True)
        sel = lane == slot0 + t
        bv = jnp.where(sel, m, bv)
        bi = jnp.where(sel, idx + j * BN, bi)
        dcur = jnp.where(iota == idx, INF, dcur)
        return dcur, bv, bi

    _, bv, bi = jax.lax.fori_loop(
        0, K, body, (d, bufv_ref[...], bufi_ref[...]))
    bufv_ref[...] = bv
    bufi_ref[...] = bi

    @pl.when((j % GROUP == GROUP - 1) | (j == nb - 1))
    def _flush():
        g = j // GROUP
        cval_ref[:, pl.ds(g * 128, 128)] = bufv_ref[...]
        cidx_ref[:, pl.ds(g * 128, 128)] = bufi_ref[...]

    @pl.when(j == nb - 1)
    def _final():
        ci = cidx_ref[...]

        def fbody(t, carry):
            cvcur, ob = carry
            m = jnp.min(cvcur, axis=1, keepdims=True)
            gi = jnp.min(jnp.where(cvcur == m, ci, BIGI), axis=1, keepdims=True)
            ob = jnp.where(lane == t, gi, ob)
            return jnp.where(ci == gi, INF, cvcur), ob

        outbuf = jnp.zeros((q, 128), jnp.int32)
        _, outbuf = jax.lax.fori_loop(0, K, fbody, (cval_ref[...], outbuf))
        out_ref[...] = outbuf[:, :K]


def kernel(s0, s1, s2, s3, y):
    size = s0.shape[0]
    a = jnp.concatenate(
        [s.reshape(size, -1) for s in (s0, s1, s2, s3)], axis=-1)
    n, d_full = y.shape
    nb = pl.cdiv(n, BN)
    ngroups = pl.cdiv(nb, GROUP)

    inds = pl.pallas_call(
        functools.partial(_dist_topk_kernel, n=n, nb=nb),
        grid=(nb,),
        in_specs=[
            pl.BlockSpec((size, d_full), lambda j: (0, 0)),
            pl.BlockSpec((BN, d_full), lambda j: (j, 0)),
        ],
        out_specs=pl.BlockSpec((size, K), lambda j: (0, 0)),
        out_shape=jax.ShapeDtypeStruct((size, K), jnp.int32),
        scratch_shapes=[
            pltpu.VMEM((size, ngroups * 128), jnp.float32),
            pltpu.VMEM((size, ngroups * 128), jnp.int32),
            pltpu.VMEM((size, 128), jnp.float32),
            pltpu.VMEM((size, 128), jnp.int32),
        ],
    )(a, y)
    return (s0, s1, s2, s3, inds)


# speedup vs baseline: 6.8888x; 4.8554x over previous
import functools

import jax
import jax.numpy as jnp
from jax.experimental import pallas as pl
from jax.experimental.pallas import tpu as pltpu

BN = 2048
LW = 256
R = 4
K = 10
INF = float("inf")
BIGI = 2**30


def _dist_topk_kernel(a_ref, y_ref, out_ref, sv_ref, si_ref, *, n, nb):
    j = pl.program_id(0)
    q = a_ref.shape[0]

    @pl.when(j == 0)
    def _init():
        sv_ref[...] = jnp.full(sv_ref.shape, INF, jnp.float32)
        si_ref[...] = jnp.full(si_ref.shape, BIGI, jnp.int32)

    a = a_ref[...]
    yb = y_ref[...]
    ab = jax.lax.dot_general(
        a.astype(jnp.bfloat16), yb.astype(jnp.bfloat16),
        (((1,), (1,)), ((), ())),
        preferred_element_type=jnp.float32)
    a2 = jnp.sum(a * a, axis=1, keepdims=True)
    b2 = jnp.sum(yb * yb, axis=1)[None, :]
    d = (a2 + b2) - 2.0 * ab
    iota = jax.lax.broadcasted_iota(jnp.int32, (q, BN), 1)
    d = jnp.where(iota < (n - j * BN), d, INF)

    vs = [sv_ref[:, pl.ds(r * LW, LW)] for r in range(R)]
    is_ = [si_ref[:, pl.ds(r * LW, LW)] for r in range(R)]
    ei0 = jax.lax.broadcasted_iota(jnp.int32, (q, LW), 1)

    for s in range(BN // LW):
        e = d[:, s * LW:(s + 1) * LW]
        ei = ei0 + (j * BN + s * LW)
        c = [e < vs[r] for r in range(R)]
        for r in range(R - 1, 0, -1):
            vs[r] = jnp.where(c[r], jnp.where(c[r - 1], vs[r - 1], e), vs[r])
            is_[r] = jnp.where(c[r], jnp.where(c[r - 1], is_[r - 1], ei), is_[r])
        vs[0] = jnp.where(c[0], e, vs[0])
        is_[0] = jnp.where(c[0], ei, is_[0])

    for r in range(R):
        sv_ref[:, pl.ds(r * LW, LW)] = vs[r]
        si_ref[:, pl.ds(r * LW, LW)] = is_[r]

    @pl.when(j == nb - 1)
    def _final():
        cv = jnp.concatenate(vs, axis=1)
        ci = jnp.concatenate(is_, axis=1)
        lane = jax.lax.broadcasted_iota(jnp.int32, (q, 128), 1)

        def fbody(t, carry):
            cvcur, ob = carry
            m = jnp.min(cvcur, axis=1, keepdims=True)
            gi = jnp.min(jnp.where(cvcur == m, ci, BIGI), axis=1, keepdims=True)
            ob = jnp.where(lane == t, gi, ob)
            return jnp.where(ci == gi, INF, cvcur), ob

        outbuf = jnp.zeros((q, 128), jnp.int32)
        _, outbuf = jax.lax.fori_loop(0, K, fbody, (cv, outbuf))
        out_ref[...] = outbuf[:, :K]


def kernel(s0, s1, s2, s3, y):
    size = s0.shape[0]
    a = jnp.concatenate(
        [s.reshape(size, -1) for s in (s0, s1, s2, s3)], axis=-1)
    n, d_full = y.shape
    nb = pl.cdiv(n, BN)

    inds = pl.pallas_call(
        functools.partial(_dist_topk_kernel, n=n, nb=nb),
        grid=(nb,),
        in_specs=[
            pl.BlockSpec((size, d_full), lambda j: (0, 0)),
            pl.BlockSpec((BN, d_full), lambda j: (j, 0)),
        ],
        out_specs=pl.BlockSpec((size, K), lambda j: (0, 0)),
        out_shape=jax.ShapeDtypeStruct((size, K), jnp.int32),
        scratch_shapes=[
            pltpu.VMEM((size, R * LW), jnp.float32),
            pltpu.VMEM((size, R * LW), jnp.int32),
        ],
    )(a, y)
    return (s0, s1, s2, s3, inds)
